# baseline (speedup 1.0000x reference)
import functools

import jax
import jax.numpy as jnp
from jax import lax
from jax.experimental import pallas as pl
from jax.experimental.pallas import tpu as pltpu
from jax.experimental.pallas import tpu_sc as plsc

N_NODES = 10000
N_PAD = 10112
N_EDGES = 320000
C_DIM = 128

NC = 2
NS = 16
NW = NC * NS
L = 16

CHUNK = 64
E_PAD = 327680
EPW = E_PAD // NW
NCHUNK = EPW // CHUNK
ROWS_PER_TILE = N_PAD // NS
NBUF = 4
G = 40
NBATCH = NCHUNK // G


def _mm_body(x_ref, w_ref, b_ref, h_ref, hb_ref):
    h = jnp.dot(x_ref[...], w_ref[...], preferred_element_type=jnp.float32)
    h_ref[...] = h
    hb_ref[...] = h + b_ref[...]


def _matmul(x, w, bias2d):
    m = x.shape[0]
    bm = 632
    return pl.pallas_call(
        _mm_body,
        grid=(m // bm,),
        in_specs=[
            pl.BlockSpec((bm, C_DIM), lambda i: (i, 0)),
            pl.BlockSpec((C_DIM, C_DIM), lambda i: (0, 0)),
            pl.BlockSpec((1, C_DIM), lambda i: (0, 0)),
        ],
        out_specs=[
            pl.BlockSpec((bm, C_DIM), lambda i: (i, 0)),
            pl.BlockSpec((bm, C_DIM), lambda i: (i, 0)),
        ],
        out_shape=[
            jax.ShapeDtypeStruct((m, C_DIM), jnp.float32),
            jax.ShapeDtypeStruct((m, C_DIM), jnp.float32),
        ],
    )(x, w, bias2d)


def _add_body(a_ref, b_ref, o_ref):
    o_ref[...] = a_ref[...] + b_ref[...]


def _final_add(a, b):
    m = N_NODES
    bm = 2000
    return pl.pallas_call(
        _add_body,
        grid=(m // bm,),
        in_specs=[
            pl.BlockSpec((bm, C_DIM), lambda i: (i, 0)),
            pl.BlockSpec((bm, C_DIM), lambda i: (i, 0)),
        ],
        out_specs=pl.BlockSpec((bm, C_DIM), lambda i: (i, 0)),
        out_shape=jax.ShapeDtypeStruct((m, C_DIM), jnp.float32),
    )(a, b)


def _sc_body(h_hbm, src_hbm, dst_hbm, d_hbm, cvec_hbm, hb_hbm, zeros_hbm,
             p0_hbm, p1_hbm,
             src_b, dst_b, d_b, rows_v, cv, accum, sem0, sem1, sem2, sem3):
    c = lax.axis_index("c")
    s = lax.axis_index("s")
    wid = s * NC + c
    cbase = wid * NCHUNK
    rbase = s * ROWS_PER_TILE
    sems = (sem0, sem1, sem2, sem3)

    @pl.when(c == 0)
    def _():
        pltpu.sync_copy(hb_hbm.at[pl.ds(rbase, ROWS_PER_TILE)],
                        accum.at[pl.ds(rbase, ROWS_PER_TILE)])

    @pl.when(c != 0)
    def _():
        pltpu.sync_copy(zeros_hbm.at[pl.ds(rbase, ROWS_PER_TILE)],
                        accum.at[pl.ds(rbase, ROWS_PER_TILE)])

    pltpu.sync_copy(cvec_hbm, cv)
    plsc.subcore_barrier()

    def start_gather(k, b):
        pltpu.make_async_copy(h_hbm.at[src_b.at[k]], rows_v.at[b],
                              sems[b]).start()

    def compute_chunk(k, b):
        pltpu.make_async_copy(h_hbm.at[src_b.at[k]], rows_v.at[b],
                              sems[b]).wait()
        cvec = cv[...]

        @plsc.parallel_loop(0, CHUNK // L, unroll=2)
        def group_body(t):
            dv = d_b[k, pl.ds(t * L, L)]
            wv = jnp.exp(dv * dv * cvec)
            for e2 in range(L):
                wb = lax.gather(
                    wv, jnp.full((L, 1), e2, jnp.int32),
                    lax.GatherDimensionNumbers(offset_dims=(),
                                               collapsed_slice_dims=(0,),
                                               start_index_map=(0,)),
                    slice_sizes=(1,),
                    mode=lax.GatherScatterMode.PROMISE_IN_BOUNDS)
                e = t * L + e2
                for j in range(C_DIM // L):
                    sl = pl.ds(j * L, L)
                    rows_v[b, e, sl] = rows_v[b, e, sl] * wb
        pltpu.sync_copy(rows_v.at[b], accum.at[dst_b.at[k]], add=True)

    def batch_body(q, carry):
        qbase = cbase + q * G
        pltpu.sync_copy(src_hbm.at[pl.ds(qbase, G)], src_b)
        pltpu.sync_copy(dst_hbm.at[pl.ds(qbase, G)], dst_b)
        pltpu.sync_copy(d_hbm.at[pl.ds(qbase, G)], d_b)
        for b in range(NBUF):
            start_gather(b, b)

        def pair_body(k2, carry2):
            for b in range(NBUF):
                k = k2 * NBUF + b
                compute_chunk(k, b)

                @pl.when(k2 < G // NBUF - 1)
                def _():
                    start_gather(k + NBUF, b)
            return carry2

        lax.fori_loop(0, G // NBUF, pair_body, 0)
        return carry

    lax.fori_loop(0, NBATCH, batch_body, 0)
    plsc.subcore_barrier()

    @pl.when(c == 0)
    def _():
        pltpu.sync_copy(accum.at[pl.ds(rbase, ROWS_PER_TILE)],
                        p0_hbm.at[pl.ds(rbase, ROWS_PER_TILE)])

    @pl.when(c != 0)
    def _():
        pltpu.sync_copy(accum.at[pl.ds(rbase, ROWS_PER_TILE)],
                        p1_hbm.at[pl.ds(rbase, ROWS_PER_TILE)])


_sc_scatter = functools.partial(
    pl.kernel,
    out_type=(
        jax.ShapeDtypeStruct((N_PAD, C_DIM), jnp.float32),
        jax.ShapeDtypeStruct((N_PAD, C_DIM), jnp.float32),
    ),
    mesh=plsc.VectorSubcoreMesh(core_axis_name="c", subcore_axis_name="s"),
    scratch_types=[
        pltpu.VMEM((G, CHUNK), jnp.int32),
        pltpu.VMEM((G, CHUNK), jnp.int32),
        pltpu.VMEM((G, CHUNK), jnp.float32),
        pltpu.VMEM((NBUF, CHUNK, C_DIM), jnp.float32),
        pltpu.VMEM((L,), jnp.float32),
        pltpu.VMEM_SHARED((N_PAD, C_DIM), jnp.float32),
        pltpu.SemaphoreType.DMA,
        pltpu.SemaphoreType.DMA,
        pltpu.SemaphoreType.DMA,
        pltpu.SemaphoreType.DMA,
    ],
)(_sc_body)


def kernel(x, edge_index, edge_attr, root_weight, bias, gaussian_param):
    src = edge_index[0].astype(jnp.int32)
    dst = edge_index[1].astype(jnp.int32)
    d = edge_attr[:, 0]
    n_pad = E_PAD - src.shape[0]
    src_p = jnp.concatenate([src, jnp.zeros((n_pad,), jnp.int32)]).reshape(-1, CHUNK)
    dst_p = jnp.concatenate([dst, jnp.zeros((n_pad,), jnp.int32)]).reshape(-1, CHUNK)
    d_p = jnp.concatenate([d, jnp.full((n_pad,), 1e30, jnp.float32)]).reshape(-1, CHUNK)
    cvec = jnp.full((L,), -1.0 / (gaussian_param[0] ** 2 + 1e-8), jnp.float32)
    zeros = jnp.zeros((N_PAD, C_DIM), jnp.float32)

    x_p = jnp.concatenate([x, jnp.zeros((N_PAD - N_NODES, C_DIM), jnp.float32)])
    h, hb = _matmul(x_p, root_weight, bias.reshape(1, C_DIM))
    p0, p1 = _sc_scatter(h, src_p, dst_p, d_p, cvec, hb, zeros)
    return _final_add(p0, p1)


# speedup vs baseline: 1.0101x; 1.0101x over previous
import functools

import jax
import jax.numpy as jnp
from jax import lax
from jax.experimental import pallas as pl
from jax.experimental.pallas import tpu as pltpu
from jax.experimental.pallas import tpu_sc as plsc

N_NODES = 10000
N_PAD = 10112
N_EDGES = 320000
C_DIM = 128

NC = 2
NS = 16
NW = NC * NS
L = 16

CHUNK = 64
E_PAD = 327680
EPW = E_PAD // NW
NCHUNK = EPW // CHUNK
ROWS_PER_TILE = N_PAD // NS
NBUF = 2
G = 80
NBATCH = NCHUNK // G


def _mm_body(x_ref, w_ref, b_ref, h_ref, hb_ref):
    h = jnp.dot(x_ref[...], w_ref[...], preferred_element_type=jnp.float32)
    h_ref[...] = h
    hb_ref[...] = h + b_ref[...]


def _matmul(x, w, bias2d):
    m = x.shape[0]
    bm = 632
    return pl.pallas_call(
        _mm_body,
        grid=(m // bm,),
        in_specs=[
            pl.BlockSpec((bm, C_DIM), lambda i: (i, 0)),
            pl.BlockSpec((C_DIM, C_DIM), lambda i: (0, 0)),
            pl.BlockSpec((1, C_DIM), lambda i: (0, 0)),
        ],
        out_specs=[
            pl.BlockSpec((bm, C_DIM), lambda i: (i, 0)),
            pl.BlockSpec((bm, C_DIM), lambda i: (i, 0)),
        ],
        out_shape=[
            jax.ShapeDtypeStruct((m, C_DIM), jnp.float32),
            jax.ShapeDtypeStruct((m, C_DIM), jnp.float32),
        ],
    )(x, w, bias2d)


def _add_body(a_ref, b_ref, o_ref):
    o_ref[...] = a_ref[...] + b_ref[...]


def _final_add(a, b):
    m = N_NODES
    bm = 2000
    return pl.pallas_call(
        _add_body,
        grid=(m // bm,),
        in_specs=[
            pl.BlockSpec((bm, C_DIM), lambda i: (i, 0)),
            pl.BlockSpec((bm, C_DIM), lambda i: (i, 0)),
        ],
        out_specs=pl.BlockSpec((bm, C_DIM), lambda i: (i, 0)),
        out_shape=jax.ShapeDtypeStruct((m, C_DIM), jnp.float32),
    )(a, b)


def _sc_body(h_hbm, src_hbm, dst_hbm, d_hbm, cvec_hbm, hb_hbm, zeros_hbm,
             p0_hbm, p1_hbm,
             src_b, dst_b, d_b, rows_v, cv, accum, sem0, sem1, sem2, sem3):
    c = lax.axis_index("c")
    s = lax.axis_index("s")
    wid = s * NC + c
    cbase = wid * NCHUNK
    rbase = s * ROWS_PER_TILE
    sems = (sem0, sem1)

    @pl.when(c == 0)
    def _():
        pltpu.sync_copy(hb_hbm.at[pl.ds(rbase, ROWS_PER_TILE)],
                        accum.at[pl.ds(rbase, ROWS_PER_TILE)])

    @pl.when(c != 0)
    def _():
        pltpu.sync_copy(zeros_hbm.at[pl.ds(rbase, ROWS_PER_TILE)],
                        accum.at[pl.ds(rbase, ROWS_PER_TILE)])

    pltpu.sync_copy(cvec_hbm, cv)
    plsc.subcore_barrier()

    def start_gather(k, b):
        for g4 in range(CHUNK // L):
            sv = src_b[k, pl.ds(g4 * L, L)]
            pltpu.make_async_copy(h_hbm.at[sv],
                                  rows_v.at[b, pl.ds(g4 * L, L)],
                                  sems[b]).start()

    def compute_chunk(k, b):
        for g4 in range(CHUNK // L):
            sv = src_b[k, pl.ds(g4 * L, L)]
            pltpu.make_async_copy(h_hbm.at[sv],
                                  rows_v.at[b, pl.ds(g4 * L, L)],
                                  sems[b]).wait()
        cvec = cv[...]

        @plsc.parallel_loop(0, CHUNK // L, unroll=2)
        def group_body(t):
            dv = d_b[k, pl.ds(t * L, L)]
            wv = jnp.exp(dv * dv * cvec)
            for e2 in range(L):
                wb = lax.gather(
                    wv, jnp.full((L, 1), e2, jnp.int32),
                    lax.GatherDimensionNumbers(offset_dims=(),
                                               collapsed_slice_dims=(0,),
                                               start_index_map=(0,)),
                    slice_sizes=(1,),
                    mode=lax.GatherScatterMode.PROMISE_IN_BOUNDS)
                e = t * L + e2
                for j in range(C_DIM // L):
                    sl = pl.ds(j * L, L)
                    rows_v[b, e, sl] = rows_v[b, e, sl] * wb
        pltpu.sync_copy(rows_v.at[b], accum.at[dst_b.at[k]], add=True)

    def batch_body(q, carry):
        qbase = cbase + q * G
        pltpu.sync_copy(src_hbm.at[pl.ds(qbase, G)], src_b)
        pltpu.sync_copy(dst_hbm.at[pl.ds(qbase, G)], dst_b)
        pltpu.sync_copy(d_hbm.at[pl.ds(qbase, G)], d_b)
        for b in range(NBUF):
            start_gather(b, b)

        def pair_body(k2, carry2):
            for b in range(NBUF):
                k = k2 * NBUF + b
                compute_chunk(k, b)

                @pl.when(k2 < G // NBUF - 1)
                def _():
                    start_gather(k + NBUF, b)
            return carry2

        lax.fori_loop(0, G // NBUF, pair_body, 0)
        return carry

    lax.fori_loop(0, NBATCH, batch_body, 0)
    plsc.subcore_barrier()

    @pl.when(c == 0)
    def _():
        pltpu.sync_copy(accum.at[pl.ds(rbase, ROWS_PER_TILE)],
                        p0_hbm.at[pl.ds(rbase, ROWS_PER_TILE)])

    @pl.when(c != 0)
    def _():
        pltpu.sync_copy(accum.at[pl.ds(rbase, ROWS_PER_TILE)],
                        p1_hbm.at[pl.ds(rbase, ROWS_PER_TILE)])


_sc_scatter = functools.partial(
    pl.kernel,
    out_type=(
        jax.ShapeDtypeStruct((N_PAD, C_DIM), jnp.float32),
        jax.ShapeDtypeStruct((N_PAD, C_DIM), jnp.float32),
    ),
    mesh=plsc.VectorSubcoreMesh(core_axis_name="c", subcore_axis_name="s"),
    scratch_types=[
        pltpu.VMEM((G, CHUNK), jnp.int32),
        pltpu.VMEM((G, CHUNK), jnp.int32),
        pltpu.VMEM((G, CHUNK), jnp.float32),
        pltpu.VMEM((NBUF, CHUNK, C_DIM), jnp.float32),
        pltpu.VMEM((L,), jnp.float32),
        pltpu.VMEM_SHARED((N_PAD, C_DIM), jnp.float32),
        pltpu.SemaphoreType.DMA,
        pltpu.SemaphoreType.DMA,
        pltpu.SemaphoreType.DMA,
        pltpu.SemaphoreType.DMA,
    ],
)(_sc_body)


def kernel(x, edge_index, edge_attr, root_weight, bias, gaussian_param):
    src = edge_index[0].astype(jnp.int32)
    dst = edge_index[1].astype(jnp.int32)
    d = edge_attr[:, 0]
    n_pad = E_PAD - src.shape[0]
    src_p = jnp.concatenate([src, jnp.zeros((n_pad,), jnp.int32)]).reshape(-1, CHUNK)
    dst_p = jnp.concatenate([dst, jnp.zeros((n_pad,), jnp.int32)]).reshape(-1, CHUNK)
    d_p = jnp.concatenate([d, jnp.full((n_pad,), 1e30, jnp.float32)]).reshape(-1, CHUNK)
    cvec = jnp.full((L,), -1.0 / (gaussian_param[0] ** 2 + 1e-8), jnp.float32)
    zeros = jnp.zeros((N_PAD, C_DIM), jnp.float32)

    x_p = jnp.concatenate([x, jnp.zeros((N_PAD - N_NODES, C_DIM), jnp.float32)])
    h, hb = _matmul(x_p, root_weight, bias.reshape(1, C_DIM))
    p0, p1 = _sc_scatter(h, src_p, dst_p, d_p, cvec, hb, zeros)
    return _final_add(p0, p1)
